# speedup vs baseline: 1.9605x; 1.9605x over previous
import functools

import jax
import jax.numpy as jnp
from jax import lax
from jax.experimental import pallas as pl
from jax.experimental.pallas import tpu as pltpu
from jax.experimental.pallas import tpu_sc as plsc

VOCAB = 32128
EMBED_DIM = 768
BATCH = 4096
SEQ = 200

NC = 2
NS = 16
NW = NC * NS

B = BATCH * SEQ
B_PER_W = B // NW
CHUNK = 64
N_CHUNKS = B_PER_W // CHUNK
NBUF = 2


def _body(table_hbm, idx_hbm, out_hbm, idx_v, rows_v, sems):
    wid = lax.axis_index("s") * NC + lax.axis_index("c")
    base = wid * B_PER_W

    pltpu.sync_copy(idx_hbm.at[pl.ds(base, B_PER_W)], idx_v)

    def start_gather(g, buf):
        idx_slice = idx_v.at[pl.ds(pl.multiple_of(g * CHUNK, 8), CHUNK)]
        pltpu.make_async_copy(table_hbm.at[idx_slice], rows_v.at[buf],
                              sems.at[buf]).start()

    def wait_gather(buf):
        pltpu.make_async_copy(table_hbm.at[idx_v.at[pl.ds(0, CHUNK)]],
                              rows_v.at[buf], sems.at[buf]).wait()

    for b in range(NBUF):
        start_gather(b, b)

    @pl.loop(0, N_CHUNKS, step=NBUF)
    def _(i):
        for b in range(NBUF):
            g = i + b
            wait_gather(b)
            pltpu.sync_copy(
                rows_v.at[b],
                out_hbm.at[pl.ds(base + pl.multiple_of(g * CHUNK, 8), CHUNK)])

            @pl.when(g + NBUF < N_CHUNKS)
            def _():
                start_gather(g + NBUF, b)


@functools.partial(
    pl.kernel,
    out_type=jax.ShapeDtypeStruct((B, EMBED_DIM), jnp.float32),
    mesh=plsc.VectorSubcoreMesh(core_axis_name="c", subcore_axis_name="s"),
    scratch_types=[
        pltpu.VMEM((B_PER_W,), jnp.int32),
        pltpu.VMEM((NBUF, CHUNK, EMBED_DIM), jnp.float32),
        pltpu.SemaphoreType.DMA((NBUF,)),
    ],
)
def _gather_kernel(table_hbm, idx_hbm, out_hbm, idx_v, rows_v, sems):
    _body(table_hbm, idx_hbm, out_hbm, idx_v, rows_v, sems)


def kernel(x, embedding_table):
    idx = x.reshape(-1).astype(jnp.int32)
    out = _gather_kernel(embedding_table, idx)
    return out.reshape(BATCH, SEQ, EMBED_DIM)
